# Initial kernel scaffold; baseline (speedup 1.0000x reference)
import jax
import jax.numpy as jnp
from jax.experimental import pallas as pl


def kernel(user_ids, item_ids, pos_item_idx, price, platform_idx, device_idx, sum_action_item_before, is_first_in_impression, list_action_type_idx, list_clickout_item_idx, list_interaction_item_image_idx, list_interaction_item_info_idx, list_interaction_item_rating_idx, list_interaction_item_deals_idx, list_search_for_item_idx, list_search_for_poi, list_change_of_sort_order, list_search_for_destination, list_filter_selection, list_current_filters, list_metadata, user_table, item_table, action_table, word_table, gru_Wih, gru_Whh, gru_bih, gru_bhh, W1, b1, W2, b2, Wout, bout):
    raise NotImplementedError("write your pallas kernel here")


# speedup vs baseline: 1.4663x; 1.4663x over previous
import functools

import jax
import jax.numpy as jnp
from jax import lax
from jax.experimental import pallas as pl
from jax.experimental.pallas import tpu as pltpu
from jax.experimental.pallas import tpu_sc as plsc

BB = 1024
NFEAT = 128
WHIST = 20
HIDDEN = 10
NITEMLISTS = 6
NSMALL = 5

_NC = 2
_NS = 16
_NW = _NC * _NS

ITEM_ROWS = BB + NITEMLISTS * BB * WHIST
ITEM_TOT = 131072
ROWS_PER_W = ITEM_TOT // _NW
CHUNK = 128
NCHUNK = ROWS_PER_W // CHUNK
U_PER_W = BB // _NW


def _sc_gather_body(item_tab, idx_item, user_tab, uid, out_item, out_user,
                    idx_v, rows_v, idxu_v, rowsu_v, sem):
    wid = lax.axis_index("s") * _NC + lax.axis_index("c")
    base = wid * ROWS_PER_W

    def chunk(c, carry):
        off = base + c * CHUNK
        pltpu.sync_copy(idx_item.at[pl.ds(off, CHUNK)], idx_v)
        pltpu.async_copy(item_tab.at[idx_v], rows_v, sem).wait()
        pltpu.sync_copy(rows_v, out_item.at[pl.ds(off, CHUNK)])
        return carry

    lax.fori_loop(0, NCHUNK, chunk, 0)

    ubase = wid * U_PER_W
    pltpu.sync_copy(uid.at[pl.ds(ubase, U_PER_W)], idxu_v)
    pltpu.async_copy(user_tab.at[idxu_v], rowsu_v, sem).wait()
    pltpu.sync_copy(rowsu_v, out_user.at[pl.ds(ubase, U_PER_W)])


def _sc_gather(item_table, idx_item, user_table, user_ids):
    mesh = plsc.VectorSubcoreMesh(core_axis_name="c", subcore_axis_name="s")
    call = functools.partial(
        pl.kernel, mesh=mesh,
        out_type=[jax.ShapeDtypeStruct((ITEM_TOT, NFEAT), jnp.float32),
                  jax.ShapeDtypeStruct((BB, NFEAT), jnp.float32)],
        scratch_types=[pltpu.VMEM((CHUNK,), jnp.int32),
                       pltpu.VMEM((CHUNK, NFEAT), jnp.float32),
                       pltpu.VMEM((U_PER_W,), jnp.int32),
                       pltpu.VMEM((U_PER_W, NFEAT), jnp.float32),
                       pltpu.SemaphoreType.DMA],
    )(_sc_gather_body)
    return call(item_table, idx_item, user_table, user_ids)


def _gru_item_body(seq_ref, wih_ref, whh_ref, bih_ref, bhh_ref, out_ref):
    wih = wih_ref[0]
    whh = whh_ref[0]
    bih = bih_ref[0]
    bhh = bhh_ref[0]
    h = jnp.zeros((BB, HIDDEN), jnp.float32)
    for t in range(WHIST):
        xt = seq_ref[0, :, t, :]
        gi = lax.dot_general(xt, wih, (((1,), (1,)), ((), ())),
                             preferred_element_type=jnp.float32) + bih
        gh = lax.dot_general(h, whh, (((1,), (1,)), ((), ())),
                             preferred_element_type=jnp.float32) + bhh
        r = jax.nn.sigmoid(gi[:, 0:HIDDEN] + gh[:, 0:HIDDEN])
        z = jax.nn.sigmoid(gi[:, HIDDEN:2 * HIDDEN] + gh[:, HIDDEN:2 * HIDDEN])
        n = jnp.tanh(gi[:, 2 * HIDDEN:] + r * gh[:, 2 * HIDDEN:])
        h = (1.0 - z) * n + z * h
        out_ref[0, t] = h


def _gru_small_body(idx_ref, tab_ref, wih_ref, whh_ref, bih_ref, bhh_ref,
                    out_ref):
    wih = wih_ref[0]
    whh = whh_ref[0]
    bih = bih_ref[0]
    bhh = bhh_ref[0]
    ptab = lax.dot_general(tab_ref[0], wih, (((1,), (1,)), ((), ())),
                           preferred_element_type=jnp.float32)
    h = jnp.zeros((BB, HIDDEN), jnp.float32)
    cols = lax.broadcasted_iota(jnp.int32, (BB, 96), 1)
    for t in range(WHIST):
        it = idx_ref[0, :, t]
        oh = (it[:, None] == cols).astype(jnp.float32)
        gi = jnp.dot(oh, ptab, preferred_element_type=jnp.float32) + bih
        gh = lax.dot_general(h, whh, (((1,), (1,)), ((), ())),
                             preferred_element_type=jnp.float32) + bhh
        r = jax.nn.sigmoid(gi[:, 0:HIDDEN] + gh[:, 0:HIDDEN])
        z = jax.nn.sigmoid(gi[:, HIDDEN:2 * HIDDEN] + gh[:, HIDDEN:2 * HIDDEN])
        n = jnp.tanh(gi[:, 2 * HIDDEN:] + r * gh[:, 2 * HIDDEN:])
        h = (1.0 - z) * n + z * h
        out_ref[0, t] = h


def _head_body(ce_ref, w1_ref, b1_ref, w2_ref, b2_ref, wout_ref, bout_ref,
               out_ref):
    h1 = jnp.maximum(
        jnp.dot(ce_ref[...], w1_ref[...],
                preferred_element_type=jnp.float32) + b1_ref[...], 0.0)
    x = jnp.dot(h1, w2_ref[...], preferred_element_type=jnp.float32) + b2_ref[...]
    o = jax.nn.sigmoid(
        jnp.dot(x, wout_ref[...], preferred_element_type=jnp.float32)
        + bout_ref[...])
    out_ref[...] = o


def kernel(user_ids, item_ids, pos_item_idx, price, platform_idx, device_idx,
           sum_action_item_before, is_first_in_impression,
           list_action_type_idx, list_clickout_item_idx,
           list_interaction_item_image_idx, list_interaction_item_info_idx,
           list_interaction_item_rating_idx, list_interaction_item_deals_idx,
           list_search_for_item_idx, list_search_for_poi,
           list_change_of_sort_order, list_search_for_destination,
           list_filter_selection, list_current_filters, list_metadata,
           user_table, item_table, action_table, word_table, gru_Wih, gru_Whh,
           gru_bih, gru_bhh, W1, b1, W2, b2, Wout, bout):
    f32 = jnp.float32

    item_lists = [list_clickout_item_idx, list_interaction_item_image_idx,
                  list_interaction_item_info_idx,
                  list_interaction_item_rating_idx,
                  list_interaction_item_deals_idx, list_search_for_item_idx]
    idx_item = jnp.concatenate(
        [item_ids] + [l.reshape(-1) for l in item_lists]
        + [jnp.zeros((ITEM_TOT - ITEM_ROWS,), jnp.int32)])
    out_item, user_emb = _sc_gather(item_table, idx_item, user_table, user_ids)
    item_emb = out_item[:BB]
    seq_item = out_item[BB:ITEM_ROWS].reshape(NITEMLISTS, BB, WHIST, NFEAT)

    bih3 = gru_bih.reshape(11, 1, 30)
    bhh3 = gru_bhh.reshape(11, 1, 30)
    ys_item = pl.pallas_call(
        _gru_item_body,
        grid=(NITEMLISTS,),
        in_specs=[
            pl.BlockSpec((1, BB, WHIST, NFEAT), lambda g: (g, 0, 0, 0)),
            pl.BlockSpec((1, 30, NFEAT), lambda g: (g + 1, 0, 0)),
            pl.BlockSpec((1, 30, HIDDEN), lambda g: (g + 1, 0, 0)),
            pl.BlockSpec((1, 1, 30), lambda g: (g + 1, 0, 0)),
            pl.BlockSpec((1, 1, 30), lambda g: (g + 1, 0, 0)),
        ],
        out_specs=pl.BlockSpec((1, WHIST, BB, HIDDEN), lambda g: (g, 0, 0, 0)),
        out_shape=jax.ShapeDtypeStruct((NITEMLISTS, WHIST, BB, HIDDEN), f32),
    )(seq_item, gru_Wih, gru_Whh, bih3, bhh3)

    idx_small = jnp.stack([list_action_type_idx, list_search_for_poi,
                           list_change_of_sort_order,
                           list_search_for_destination,
                           list_filter_selection])
    tab_a = jnp.zeros((96, NFEAT), f32).at[:11].set(action_table)
    tab_w = jnp.zeros((96, NFEAT), f32).at[:88].set(word_table)
    tabs = jnp.stack([tab_a, tab_w, tab_w, tab_w, tab_w])
    sm = jnp.array([0, 7, 8, 9, 10], jnp.int32)
    wih_s = gru_Wih[sm]
    whh_s = gru_Whh[sm]
    bih_s = bih3[sm]
    bhh_s = bhh3[sm]
    ys_small = pl.pallas_call(
        _gru_small_body,
        grid=(NSMALL,),
        in_specs=[
            pl.BlockSpec((1, BB, WHIST), lambda g: (g, 0, 0)),
            pl.BlockSpec((1, 96, NFEAT), lambda g: (g, 0, 0)),
            pl.BlockSpec((1, 30, NFEAT), lambda g: (g, 0, 0)),
            pl.BlockSpec((1, 30, HIDDEN), lambda g: (g, 0, 0)),
            pl.BlockSpec((1, 1, 30), lambda g: (g, 0, 0)),
            pl.BlockSpec((1, 1, 30), lambda g: (g, 0, 0)),
        ],
        out_specs=pl.BlockSpec((1, WHIST, BB, HIDDEN), lambda g: (g, 0, 0, 0)),
        out_shape=jax.ShapeDtypeStruct((NSMALL, WHIST, BB, HIDDEN), f32),
    )(idx_small, tabs, wih_s, whh_s, bih_s, bhh_s)

    ys = jnp.concatenate([ys_small[0:1], ys_item, ys_small[1:5]], axis=0)
    session = ys.transpose(2, 0, 1, 3).reshape(BB, 11 * WHIST * HIDDEN)

    ce = jnp.concatenate(
        [user_emb, item_emb, sum_action_item_before[:, None],
         is_first_in_impression[:, None], pos_item_idx[:, None], session,
         list_metadata, list_current_filters, price[:, None]], axis=1)
    ND = ce.shape[1]
    NDP = 2688
    NH = W1.shape[1]
    NHP = 1408
    cep = jnp.zeros((BB, NDP), f32).at[:, :ND].set(ce)
    w1p = jnp.zeros((NDP, NHP), f32).at[:ND, :NH].set(W1)
    b1p = jnp.zeros((1, NHP), f32).at[0, :NH].set(b1)
    w2p = jnp.zeros((NHP, NFEAT), f32).at[:NH].set(W2)
    b2p = b2.reshape(1, NFEAT)
    boutp = bout.reshape(1, 1)
    BT = 128
    out = pl.pallas_call(
        _head_body,
        grid=(BB // BT,),
        in_specs=[
            pl.BlockSpec((BT, NDP), lambda g: (g, 0)),
            pl.BlockSpec((NDP, NHP), lambda g: (0, 0)),
            pl.BlockSpec((1, NHP), lambda g: (0, 0)),
            pl.BlockSpec((NHP, NFEAT), lambda g: (0, 0)),
            pl.BlockSpec((1, NFEAT), lambda g: (0, 0)),
            pl.BlockSpec((NFEAT, 1), lambda g: (0, 0)),
            pl.BlockSpec((1, 1), lambda g: (0, 0)),
        ],
        out_specs=pl.BlockSpec((BT, 1), lambda g: (g, 0)),
        out_shape=jax.ShapeDtypeStruct((BB, 1), f32),
    )(cep, w1p, b1p, w2p, b2p, Wout, boutp)
    return out


# speedup vs baseline: 1.6668x; 1.1368x over previous
import functools

import jax
import jax.numpy as jnp
from jax import lax
from jax.experimental import pallas as pl
from jax.experimental.pallas import tpu as pltpu
from jax.experimental.pallas import tpu_sc as plsc

BB = 1024
NFEAT = 128
WHIST = 20
HIDDEN = 10
NITEMLISTS = 6
NSMALL = 5

_NC = 2
_NS = 16
_NW = _NC * _NS

ITEM_ROWS = BB + NITEMLISTS * BB * WHIST
ITEM_TOT = 131072
ROWS_PER_W = ITEM_TOT // _NW
CHUNK = 128
NCHUNK = ROWS_PER_W // CHUNK
U_PER_W = BB // _NW


def _sc_gather_body(item_tab, idx_item2, user_tab, uid, out_item, out_user,
                    idxb, rows0, rows1, idxu_v, rowsu_v, sem0, sem1, semu):
    wid = lax.axis_index("s") * _NC + lax.axis_index("c")
    base = wid * ROWS_PER_W
    pltpu.sync_copy(idx_item2.at[pl.ds(wid * NCHUNK, NCHUNK)], idxb)
    pltpu.sync_copy(uid.at[pl.ds(wid * U_PER_W, U_PER_W)], idxu_v)
    pltpu.async_copy(user_tab.at[idxu_v], rowsu_v, semu)
    pltpu.async_copy(item_tab.at[idxb.at[0]], rows0, sem0)
    nhalf = NCHUNK // 2

    def body(g, carry):
        c0 = 2 * g
        pltpu.async_copy(item_tab.at[idxb.at[c0 + 1]], rows1, sem1)
        pltpu.make_async_copy(item_tab.at[idxb.at[c0]], rows0, sem0).wait()
        pltpu.sync_copy(rows0, out_item.at[pl.ds(base + c0 * CHUNK, CHUNK)])

        @pl.when(g + 1 < nhalf)
        def _():
            pltpu.async_copy(item_tab.at[idxb.at[c0 + 2]], rows0, sem0)

        pltpu.make_async_copy(item_tab.at[idxb.at[c0 + 1]], rows1, sem1).wait()
        pltpu.sync_copy(rows1,
                        out_item.at[pl.ds(base + (c0 + 1) * CHUNK, CHUNK)])
        return carry

    lax.fori_loop(0, nhalf, body, 0)
    pltpu.make_async_copy(user_tab.at[idxu_v], rowsu_v, semu).wait()
    pltpu.sync_copy(rowsu_v, out_user.at[pl.ds(wid * U_PER_W, U_PER_W)])


def _sc_gather(item_table, idx_item, user_table, user_ids):
    mesh = plsc.VectorSubcoreMesh(core_axis_name="c", subcore_axis_name="s")
    call = functools.partial(
        pl.kernel, mesh=mesh,
        out_type=[jax.ShapeDtypeStruct((ITEM_TOT, NFEAT), jnp.float32),
                  jax.ShapeDtypeStruct((BB, NFEAT), jnp.float32)],
        scratch_types=[pltpu.VMEM((NCHUNK, CHUNK), jnp.int32),
                       pltpu.VMEM((CHUNK, NFEAT), jnp.float32),
                       pltpu.VMEM((CHUNK, NFEAT), jnp.float32),
                       pltpu.VMEM((U_PER_W,), jnp.int32),
                       pltpu.VMEM((U_PER_W, NFEAT), jnp.float32),
                       pltpu.SemaphoreType.DMA,
                       pltpu.SemaphoreType.DMA,
                       pltpu.SemaphoreType.DMA],
    )(_sc_gather_body)
    return call(item_table, idx_item.reshape(ITEM_TOT // CHUNK, CHUNK),
                user_table, user_ids)


def _gru_item_body(seq_ref, wih_ref, whh_ref, bih_ref, bhh_ref, out_ref):
    wih = wih_ref[0]
    whh = whh_ref[0]
    bih = bih_ref[0]
    bhh = bhh_ref[0]
    h = jnp.zeros((BB, HIDDEN), jnp.float32)
    for t in range(WHIST):
        xt = seq_ref[0, :, t, :]
        gi = lax.dot_general(xt, wih, (((1,), (1,)), ((), ())),
                             preferred_element_type=jnp.float32) + bih
        gh = lax.dot_general(h, whh, (((1,), (1,)), ((), ())),
                             preferred_element_type=jnp.float32) + bhh
        r = jax.nn.sigmoid(gi[:, 0:HIDDEN] + gh[:, 0:HIDDEN])
        z = jax.nn.sigmoid(gi[:, HIDDEN:2 * HIDDEN] + gh[:, HIDDEN:2 * HIDDEN])
        n = jnp.tanh(gi[:, 2 * HIDDEN:] + r * gh[:, 2 * HIDDEN:])
        h = (1.0 - z) * n + z * h
        out_ref[0, t] = h


def _gru_small_body(idx_ref, tab_ref, wih_ref, whh_ref, bih_ref, bhh_ref,
                    out_ref):
    wih = wih_ref[0]
    whh = whh_ref[0]
    bih = bih_ref[0]
    bhh = bhh_ref[0]
    ptab = lax.dot_general(tab_ref[0], wih, (((1,), (1,)), ((), ())),
                           preferred_element_type=jnp.float32)
    h = jnp.zeros((BB, HIDDEN), jnp.float32)
    cols = lax.broadcasted_iota(jnp.int32, (BB, 96), 1)
    for t in range(WHIST):
        it = idx_ref[0, :, t]
        oh = (it[:, None] == cols).astype(jnp.float32)
        gi = jnp.dot(oh, ptab, preferred_element_type=jnp.float32) + bih
        gh = lax.dot_general(h, whh, (((1,), (1,)), ((), ())),
                             preferred_element_type=jnp.float32) + bhh
        r = jax.nn.sigmoid(gi[:, 0:HIDDEN] + gh[:, 0:HIDDEN])
        z = jax.nn.sigmoid(gi[:, HIDDEN:2 * HIDDEN] + gh[:, HIDDEN:2 * HIDDEN])
        n = jnp.tanh(gi[:, 2 * HIDDEN:] + r * gh[:, 2 * HIDDEN:])
        h = (1.0 - z) * n + z * h
        out_ref[0, t] = h


def _head_body(ce_ref, w1_ref, b1_ref, w2_ref, b2_ref, wout_ref, bout_ref,
               out_ref):
    h1 = jnp.maximum(
        jnp.dot(ce_ref[...], w1_ref[...],
                preferred_element_type=jnp.float32) + b1_ref[...], 0.0)
    x = jnp.dot(h1, w2_ref[...], preferred_element_type=jnp.float32) + b2_ref[...]
    o = jax.nn.sigmoid(
        jnp.dot(x, wout_ref[...], preferred_element_type=jnp.float32)
        + bout_ref[...])
    out_ref[...] = o


def kernel(user_ids, item_ids, pos_item_idx, price, platform_idx, device_idx,
           sum_action_item_before, is_first_in_impression,
           list_action_type_idx, list_clickout_item_idx,
           list_interaction_item_image_idx, list_interaction_item_info_idx,
           list_interaction_item_rating_idx, list_interaction_item_deals_idx,
           list_search_for_item_idx, list_search_for_poi,
           list_change_of_sort_order, list_search_for_destination,
           list_filter_selection, list_current_filters, list_metadata,
           user_table, item_table, action_table, word_table, gru_Wih, gru_Whh,
           gru_bih, gru_bhh, W1, b1, W2, b2, Wout, bout):
    f32 = jnp.float32

    item_lists = [list_clickout_item_idx, list_interaction_item_image_idx,
                  list_interaction_item_info_idx,
                  list_interaction_item_rating_idx,
                  list_interaction_item_deals_idx, list_search_for_item_idx]
    idx_item = jnp.concatenate(
        [item_ids] + [l.reshape(-1) for l in item_lists]
        + [jnp.zeros((ITEM_TOT - ITEM_ROWS,), jnp.int32)])
    out_item, user_emb = _sc_gather(item_table, idx_item, user_table, user_ids)
    item_emb = out_item[:BB]
    seq_item = out_item[BB:ITEM_ROWS].reshape(NITEMLISTS, BB, WHIST, NFEAT)

    bih3 = gru_bih.reshape(11, 1, 30)
    bhh3 = gru_bhh.reshape(11, 1, 30)
    ys_item = pl.pallas_call(
        _gru_item_body,
        grid=(NITEMLISTS,),
        in_specs=[
            pl.BlockSpec((1, BB, WHIST, NFEAT), lambda g: (g, 0, 0, 0)),
            pl.BlockSpec((1, 30, NFEAT), lambda g: (g + 1, 0, 0)),
            pl.BlockSpec((1, 30, HIDDEN), lambda g: (g + 1, 0, 0)),
            pl.BlockSpec((1, 1, 30), lambda g: (g + 1, 0, 0)),
            pl.BlockSpec((1, 1, 30), lambda g: (g + 1, 0, 0)),
        ],
        out_specs=pl.BlockSpec((1, WHIST, BB, HIDDEN), lambda g: (g, 0, 0, 0)),
        out_shape=jax.ShapeDtypeStruct((NITEMLISTS, WHIST, BB, HIDDEN), f32),
    )(seq_item, gru_Wih, gru_Whh, bih3, bhh3)

    idx_small = jnp.stack([list_action_type_idx, list_search_for_poi,
                           list_change_of_sort_order,
                           list_search_for_destination,
                           list_filter_selection])
    tab_a = jnp.zeros((96, NFEAT), f32).at[:11].set(action_table)
    tab_w = jnp.zeros((96, NFEAT), f32).at[:88].set(word_table)
    tabs = jnp.stack([tab_a, tab_w, tab_w, tab_w, tab_w])
    sm = jnp.array([0, 7, 8, 9, 10], jnp.int32)
    wih_s = gru_Wih[sm]
    whh_s = gru_Whh[sm]
    bih_s = bih3[sm]
    bhh_s = bhh3[sm]
    ys_small = pl.pallas_call(
        _gru_small_body,
        grid=(NSMALL,),
        in_specs=[
            pl.BlockSpec((1, BB, WHIST), lambda g: (g, 0, 0)),
            pl.BlockSpec((1, 96, NFEAT), lambda g: (g, 0, 0)),
            pl.BlockSpec((1, 30, NFEAT), lambda g: (g, 0, 0)),
            pl.BlockSpec((1, 30, HIDDEN), lambda g: (g, 0, 0)),
            pl.BlockSpec((1, 1, 30), lambda g: (g, 0, 0)),
            pl.BlockSpec((1, 1, 30), lambda g: (g, 0, 0)),
        ],
        out_specs=pl.BlockSpec((1, WHIST, BB, HIDDEN), lambda g: (g, 0, 0, 0)),
        out_shape=jax.ShapeDtypeStruct((NSMALL, WHIST, BB, HIDDEN), f32),
    )(idx_small, tabs, wih_s, whh_s, bih_s, bhh_s)

    ys = jnp.concatenate([ys_small[0:1], ys_item, ys_small[1:5]], axis=0)
    session = ys.transpose(2, 0, 1, 3).reshape(BB, 11 * WHIST * HIDDEN)

    ce = jnp.concatenate(
        [user_emb, item_emb, sum_action_item_before[:, None],
         is_first_in_impression[:, None], pos_item_idx[:, None], session,
         list_metadata, list_current_filters, price[:, None]], axis=1)
    ND = ce.shape[1]
    NH = W1.shape[1]
    BT = 128
    out = pl.pallas_call(
        _head_body,
        grid=(BB // BT,),
        in_specs=[
            pl.BlockSpec((BT, ND), lambda g: (g, 0)),
            pl.BlockSpec((ND, NH), lambda g: (0, 0)),
            pl.BlockSpec((1, NH), lambda g: (0, 0)),
            pl.BlockSpec((NH, NFEAT), lambda g: (0, 0)),
            pl.BlockSpec((1, NFEAT), lambda g: (0, 0)),
            pl.BlockSpec((NFEAT, 1), lambda g: (0, 0)),
            pl.BlockSpec((1, 1), lambda g: (0, 0)),
        ],
        out_specs=pl.BlockSpec((BT, 1), lambda g: (g, 0)),
        out_shape=jax.ShapeDtypeStruct((BB, 1), f32),
    )(ce, W1, b1.reshape(1, NH), W2, b2.reshape(1, NFEAT), Wout,
      bout.reshape(1, 1))
    return out


# speedup vs baseline: 2.0916x; 1.2549x over previous
import functools

import jax
import jax.numpy as jnp
from jax import lax
from jax.experimental import pallas as pl
from jax.experimental.pallas import tpu as pltpu
from jax.experimental.pallas import tpu_sc as plsc

BB = 1024
NFEAT = 128
WHIST = 20
HIDDEN = 10
NITEMLISTS = 6
NSMALL = 5

_NC = 2
_NS = 16
_NW = _NC * _NS

SEQ_ROWS = NITEMLISTS * BB * WHIST
ROWS_PER_W = SEQ_ROWS // _NW
CHUNK = 128
NCHUNK = ROWS_PER_W // CHUNK
U_PER_W = BB // _NW


def _sc_gather_body(item_tab, idx_seq2, iid, user_tab, uid,
                    out_seq, out_emb, out_user,
                    idxb, rows0, rows1, idxu_v, rowsu_v, idxi_v, rowsi_v,
                    sem0, sem1, semu, semi):
    wid = lax.axis_index("s") * _NC + lax.axis_index("c")
    base = wid * ROWS_PER_W
    pltpu.sync_copy(idx_seq2.at[wid], idxb)
    pltpu.sync_copy(iid.at[pl.ds(wid * U_PER_W, U_PER_W)], idxi_v)
    pltpu.async_copy(item_tab.at[idxi_v], rowsi_v, semi)
    pltpu.sync_copy(uid.at[pl.ds(wid * U_PER_W, U_PER_W)], idxu_v)
    pltpu.async_copy(user_tab.at[idxu_v], rowsu_v, semu)
    pltpu.async_copy(item_tab.at[idxb.at[0]], rows0, sem0)
    nhalf = NCHUNK // 2

    def body(g, carry):
        c0 = 2 * g
        pltpu.async_copy(item_tab.at[idxb.at[c0 + 1]], rows1, sem1)
        pltpu.make_async_copy(item_tab.at[idxb.at[c0]], rows0, sem0).wait()
        pltpu.sync_copy(rows0, out_seq.at[pl.ds(base + c0 * CHUNK, CHUNK)])

        @pl.when(g + 1 < nhalf)
        def _():
            pltpu.async_copy(item_tab.at[idxb.at[c0 + 2]], rows0, sem0)

        pltpu.make_async_copy(item_tab.at[idxb.at[c0 + 1]], rows1, sem1).wait()
        pltpu.sync_copy(rows1,
                        out_seq.at[pl.ds(base + (c0 + 1) * CHUNK, CHUNK)])
        return carry

    lax.fori_loop(0, nhalf, body, 0)
    pltpu.make_async_copy(item_tab.at[idxi_v], rowsi_v, semi).wait()
    pltpu.sync_copy(rowsi_v, out_emb.at[pl.ds(wid * U_PER_W, U_PER_W)])
    pltpu.make_async_copy(user_tab.at[idxu_v], rowsu_v, semu).wait()
    pltpu.sync_copy(rowsu_v, out_user.at[pl.ds(wid * U_PER_W, U_PER_W)])


def _sc_gather(item_table, idx_seq, item_ids, user_table, user_ids):
    mesh = plsc.VectorSubcoreMesh(core_axis_name="c", subcore_axis_name="s")
    call = functools.partial(
        pl.kernel, mesh=mesh,
        out_type=[jax.ShapeDtypeStruct((SEQ_ROWS, NFEAT), jnp.float32),
                  jax.ShapeDtypeStruct((BB, NFEAT), jnp.float32),
                  jax.ShapeDtypeStruct((BB, NFEAT), jnp.float32)],
        scratch_types=[pltpu.VMEM((NCHUNK, CHUNK), jnp.int32),
                       pltpu.VMEM((CHUNK, NFEAT), jnp.float32),
                       pltpu.VMEM((CHUNK, NFEAT), jnp.float32),
                       pltpu.VMEM((U_PER_W,), jnp.int32),
                       pltpu.VMEM((U_PER_W, NFEAT), jnp.float32),
                       pltpu.VMEM((U_PER_W,), jnp.int32),
                       pltpu.VMEM((U_PER_W, NFEAT), jnp.float32),
                       pltpu.SemaphoreType.DMA,
                       pltpu.SemaphoreType.DMA,
                       pltpu.SemaphoreType.DMA,
                       pltpu.SemaphoreType.DMA],
    )(_sc_gather_body)
    return call(item_table, idx_seq.reshape(_NW, NCHUNK, CHUNK),
                item_ids, user_table, user_ids)


def _gru_item_body(seq_ref, wih_ref, whh_ref, bih_ref, bhh_ref, out_ref):
    wih = wih_ref[0]
    whh = whh_ref[0]
    bih = bih_ref[0]
    bhh = bhh_ref[0]
    h = jnp.zeros((BB, HIDDEN), jnp.float32)
    for t in range(WHIST):
        xt = seq_ref[0, :, t, :]
        gi = lax.dot_general(xt, wih, (((1,), (1,)), ((), ())),
                             preferred_element_type=jnp.float32) + bih
        gh = lax.dot_general(h, whh, (((1,), (1,)), ((), ())),
                             preferred_element_type=jnp.float32) + bhh
        r = jax.nn.sigmoid(gi[:, 0:HIDDEN] + gh[:, 0:HIDDEN])
        z = jax.nn.sigmoid(gi[:, HIDDEN:2 * HIDDEN] + gh[:, HIDDEN:2 * HIDDEN])
        n = jnp.tanh(gi[:, 2 * HIDDEN:] + r * gh[:, 2 * HIDDEN:])
        h = (1.0 - z) * n + z * h
        out_ref[:, 0, 0, t * HIDDEN:(t + 1) * HIDDEN] = h


def _gru_small_body(idx_ref, tab_ref, wih_ref, whh_ref, bih_ref, bhh_ref,
                    out_ref):
    wih = wih_ref[0]
    whh = whh_ref[0]
    bih = bih_ref[0]
    bhh = bhh_ref[0]
    ptab = lax.dot_general(tab_ref[0], wih, (((1,), (1,)), ((), ())),
                           preferred_element_type=jnp.float32)
    h = jnp.zeros((BB, HIDDEN), jnp.float32)
    cols = lax.broadcasted_iota(jnp.int32, (BB, 96), 1)
    for t in range(WHIST):
        it = idx_ref[0, :, t]
        oh = (it[:, None] == cols).astype(jnp.float32)
        gi = jnp.dot(oh, ptab, preferred_element_type=jnp.float32) + bih
        gh = lax.dot_general(h, whh, (((1,), (1,)), ((), ())),
                             preferred_element_type=jnp.float32) + bhh
        r = jax.nn.sigmoid(gi[:, 0:HIDDEN] + gh[:, 0:HIDDEN])
        z = jax.nn.sigmoid(gi[:, HIDDEN:2 * HIDDEN] + gh[:, HIDDEN:2 * HIDDEN])
        n = jnp.tanh(gi[:, 2 * HIDDEN:] + r * gh[:, 2 * HIDDEN:])
        h = (1.0 - z) * n + z * h
        out_ref[:, 0, 0, t * HIDDEN:(t + 1) * HIDDEN] = h


def _head_body(ce_ref, w1_ref, b1_ref, w2_ref, b2_ref, wout_ref, bout_ref,
               out_ref):
    h1 = jnp.maximum(
        jnp.dot(ce_ref[...], w1_ref[...],
                preferred_element_type=jnp.float32) + b1_ref[...], 0.0)
    x = jnp.dot(h1, w2_ref[...], preferred_element_type=jnp.float32) + b2_ref[...]
    o = jax.nn.sigmoid(
        jnp.dot(x, wout_ref[...], preferred_element_type=jnp.float32)
        + bout_ref[...])
    out_ref[...] = o


def kernel(user_ids, item_ids, pos_item_idx, price, platform_idx, device_idx,
           sum_action_item_before, is_first_in_impression,
           list_action_type_idx, list_clickout_item_idx,
           list_interaction_item_image_idx, list_interaction_item_info_idx,
           list_interaction_item_rating_idx, list_interaction_item_deals_idx,
           list_search_for_item_idx, list_search_for_poi,
           list_change_of_sort_order, list_search_for_destination,
           list_filter_selection, list_current_filters, list_metadata,
           user_table, item_table, action_table, word_table, gru_Wih, gru_Whh,
           gru_bih, gru_bhh, W1, b1, W2, b2, Wout, bout):
    f32 = jnp.float32

    item_lists = [list_clickout_item_idx, list_interaction_item_image_idx,
                  list_interaction_item_info_idx,
                  list_interaction_item_rating_idx,
                  list_interaction_item_deals_idx, list_search_for_item_idx]
    idx_seq = jnp.concatenate([l.reshape(-1) for l in item_lists])
    out_seq, item_emb, user_emb = _sc_gather(item_table, idx_seq, item_ids,
                                             user_table, user_ids)
    seq_item = out_seq.reshape(NITEMLISTS, BB, WHIST, NFEAT)

    bih3 = gru_bih.reshape(11, 1, 30)
    bhh3 = gru_bhh.reshape(11, 1, 30)
    ys_item = pl.pallas_call(
        _gru_item_body,
        grid=(NITEMLISTS,),
        in_specs=[
            pl.BlockSpec((1, BB, WHIST, NFEAT), lambda g: (g, 0, 0, 0)),
            pl.BlockSpec((1, 30, NFEAT), lambda g: (g + 1, 0, 0)),
            pl.BlockSpec((1, 30, HIDDEN), lambda g: (g + 1, 0, 0)),
            pl.BlockSpec((1, 1, 30), lambda g: (g + 1, 0, 0)),
            pl.BlockSpec((1, 1, 30), lambda g: (g + 1, 0, 0)),
        ],
        out_specs=pl.BlockSpec((BB, 1, 1, WHIST * HIDDEN),
                               lambda g: (0, g, 0, 0)),
        out_shape=jax.ShapeDtypeStruct((BB, NITEMLISTS, 1, WHIST * HIDDEN),
                                       f32),
    )(seq_item, gru_Wih, gru_Whh, bih3, bhh3)

    idx_small = jnp.stack([list_action_type_idx, list_search_for_poi,
                           list_change_of_sort_order,
                           list_search_for_destination,
                           list_filter_selection])
    tab_a = jnp.zeros((96, NFEAT), f32).at[:11].set(action_table)
    tab_w = jnp.zeros((96, NFEAT), f32).at[:88].set(word_table)
    tabs = jnp.stack([tab_a, tab_w, tab_w, tab_w, tab_w])
    sm = jnp.array([0, 7, 8, 9, 10], jnp.int32)
    wih_s = gru_Wih[sm]
    whh_s = gru_Whh[sm]
    bih_s = bih3[sm]
    bhh_s = bhh3[sm]
    ys_small = pl.pallas_call(
        _gru_small_body,
        grid=(NSMALL,),
        in_specs=[
            pl.BlockSpec((1, BB, WHIST), lambda g: (g, 0, 0)),
            pl.BlockSpec((1, 96, NFEAT), lambda g: (g, 0, 0)),
            pl.BlockSpec((1, 30, NFEAT), lambda g: (g, 0, 0)),
            pl.BlockSpec((1, 30, HIDDEN), lambda g: (g, 0, 0)),
            pl.BlockSpec((1, 1, 30), lambda g: (g, 0, 0)),
            pl.BlockSpec((1, 1, 30), lambda g: (g, 0, 0)),
        ],
        out_specs=pl.BlockSpec((BB, 1, 1, WHIST * HIDDEN),
                               lambda g: (0, g, 0, 0)),
        out_shape=jax.ShapeDtypeStruct((BB, NSMALL, 1, WHIST * HIDDEN), f32),
    )(idx_small, tabs, wih_s, whh_s, bih_s, bhh_s)

    sess_item = ys_item.reshape(BB, NITEMLISTS * WHIST * HIDDEN)
    sess_small = ys_small.reshape(BB, NSMALL * WHIST * HIDDEN)

    ce = jnp.concatenate(
        [user_emb, item_emb, sum_action_item_before[:, None],
         is_first_in_impression[:, None], pos_item_idx[:, None],
         sess_small[:, :WHIST * HIDDEN], sess_item,
         sess_small[:, WHIST * HIDDEN:],
         list_metadata, list_current_filters, price[:, None]], axis=1)
    ND = ce.shape[1]
    NH = W1.shape[1]
    BT = 128
    out = pl.pallas_call(
        _head_body,
        grid=(BB // BT,),
        in_specs=[
            pl.BlockSpec((BT, ND), lambda g: (g, 0)),
            pl.BlockSpec((ND, NH), lambda g: (0, 0)),
            pl.BlockSpec((1, NH), lambda g: (0, 0)),
            pl.BlockSpec((NH, NFEAT), lambda g: (0, 0)),
            pl.BlockSpec((1, NFEAT), lambda g: (0, 0)),
            pl.BlockSpec((NFEAT, 1), lambda g: (0, 0)),
            pl.BlockSpec((1, 1), lambda g: (0, 0)),
        ],
        out_specs=pl.BlockSpec((BT, 1), lambda g: (g, 0)),
        out_shape=jax.ShapeDtypeStruct((BB, 1), f32),
    )(ce, W1, b1.reshape(1, NH), W2, b2.reshape(1, NFEAT), Wout,
      bout.reshape(1, 1))
    return out
